# baseline (speedup 1.0000x reference)
import functools

import jax
import jax.numpy as jnp
from jax import lax
from jax.experimental import pallas as pl
from jax.experimental.pallas import tpu as pltpu
from jax.experimental.pallas import tpu_sc as plsc

VOCAB = 1000000
B = 16384
OP_DIM = 32
STATS_IN = 4
STATS_H = 16
PRED_DIM = 8
OUT_DIM = 64

_NC = 2
_NS = 16
_NW = _NC * _NS
_BPW = B // _NW
_CHUNK = 128
_NCHUNK = _BPW // _CHUNK


def _sc_gather(table, idx):
    mesh = plsc.VectorSubcoreMesh(core_axis_name="c", subcore_axis_name="s")

    @functools.partial(
        pl.kernel,
        mesh=mesh,
        out_type=jax.ShapeDtypeStruct((B, OP_DIM), jnp.float32),
        scratch_types=[
            pltpu.VMEM((_NCHUNK, _CHUNK), jnp.int32),
            pltpu.VMEM((_BPW, OP_DIM), jnp.float32),
            pltpu.SemaphoreType.DMA,
        ],
        compiler_params=pltpu.CompilerParams(use_tc_tiling_on_sc=False),
    )
    def k(table_hbm, idx_hbm, out_hbm, idx_v, rows_v, sem):
        wid = lax.axis_index("s") * _NC + lax.axis_index("c")
        base = wid * _BPW
        pltpu.sync_copy(idx_hbm.at[pl.ds(wid * _NCHUNK, _NCHUNK)], idx_v)
        copies = []
        for j in range(_NCHUNK):
            copies.append(
                pltpu.async_copy(
                    table_hbm.at[idx_v.at[j]],
                    rows_v.at[pl.ds(j * _CHUNK, _CHUNK)],
                    sem,
                )
            )
        for c in copies:
            c.wait()
        pltpu.sync_copy(rows_v, out_hbm.at[pl.ds(base, _BPW)])

    return k(table, idx.reshape(_NW * _NCHUNK, _CHUNK))


def _tc_body(g_ref, s_ref, p_ref, W1_ref, b1_ref, W2_ref, b2_ref,
             Wout_ref, bout_ref, o_ref):
    s = jnp.log1p(s_ref[...])
    h = jnp.maximum(
        jnp.dot(s, W1_ref[...], preferred_element_type=jnp.float32)
        + b1_ref[...], 0.0)
    sv = (jnp.dot(h, W2_ref[...], preferred_element_type=jnp.float32)
          + b2_ref[...])
    out = (jnp.dot(g_ref[...], Wout_ref[0:OP_DIM, :],
                   preferred_element_type=jnp.float32)
           + jnp.dot(sv, Wout_ref[OP_DIM:OP_DIM + STATS_H, :],
                     preferred_element_type=jnp.float32)
           + jnp.dot(p_ref[...], Wout_ref[OP_DIM + STATS_H:, :],
                     preferred_element_type=jnp.float32)
           + bout_ref[...])
    o_ref[...] = out


def _tc_dense(gathered, stats, pred_feat, W1, b1, W2, b2, Wout, bout):
    BLK = 2048
    grid = (B // BLK,)
    row_spec = lambda d: pl.BlockSpec((BLK, d), lambda i: (i, 0))
    full = lambda a: pl.BlockSpec(a.shape, lambda i: tuple(0 for _ in a.shape))
    b1_2d = b1.reshape(1, STATS_H)
    b2_2d = b2.reshape(1, STATS_H)
    bout_2d = bout.reshape(1, OUT_DIM)
    return pl.pallas_call(
        _tc_body,
        grid=grid,
        in_specs=[
            row_spec(OP_DIM),
            row_spec(STATS_IN),
            row_spec(PRED_DIM),
            full(W1), full(b1_2d), full(W2), full(b2_2d),
            full(Wout), full(bout_2d),
        ],
        out_specs=pl.BlockSpec((BLK, OUT_DIM), lambda i: (i, 0)),
        out_shape=jax.ShapeDtypeStruct((B, OUT_DIM), jnp.float32),
    )(gathered, stats, pred_feat, W1, b1_2d, W2, b2_2d, Wout, bout_2d)


def kernel(op_idx, stats, pred_feat, emb_table, W1, b1, W2, b2, Wout, bout):
    gathered = _sc_gather(emb_table, op_idx.astype(jnp.int32))
    return _tc_dense(gathered, stats, pred_feat, W1, b1, W2, b2, Wout, bout)
